# Initial kernel scaffold; baseline (speedup 1.0000x reference)
import jax
import jax.numpy as jnp
from jax.experimental import pallas as pl


def kernel(token_ids, tabla_cuant):
    raise NotImplementedError("write your pallas kernel here")


# speedup vs baseline: 7.0945x; 7.0945x over previous
import jax
import jax.numpy as jnp
from jax import lax
from jax.experimental import pallas as pl
from jax.experimental.pallas import tpu as pltpu
from jax.experimental.pallas import tpu_sc as plsc

VOCAB = 1000000
N_ZONAS = 52
ROW_WORDS = 8
B = 4096
L = 200
TOKENS = B * L
NUM_WORKERS = 32
PER_WORKER = TOKENS // NUM_WORKERS
CHUNK = 1024
CHUNKS = PER_WORKER // CHUNK
GATHER_SPLIT = CHUNK // 128


def _sc_kernel(table_hbm, ids_hbm, out_hbm, ids_v, idx_v, rows_v, out_v, sem):
    wid = lax.axis_index("s") * 2 + lax.axis_index("c")
    iota16 = lax.iota(jnp.int32, 16)

    def chunk_body(c, carry):
        base = (wid * CHUNKS + c) * CHUNK
        pltpu.sync_copy(ids_hbm.at[pl.ds(base, CHUNK)], ids_v)
        for j in range(GATHER_SPLIT):
            for k in range(8):
                idx_v[j, pl.ds(k * 16, 16)] = ids_v[pl.ds(j * 128 + k * 16, 16)]
        descs = []
        for j in range(GATHER_SPLIT):
            d = pltpu.async_copy(
                table_hbm.at[idx_v.at[j]],
                rows_v.at[pl.ds(j * 128, 128)],
                sem,
            )
            descs.append(d)
        for d in descs:
            d.wait()

        def group_body(g, carry2):
            rid = g * 16 + iota16
            out_base = rid * N_ZONAS
            for w in range(7):
                col = jnp.full((16,), w, jnp.int32)
                val = plsc.load_gather(rows_v, [rid, col])
                n_nib = 8 if w < 6 else 4
                for n in range(n_nib):
                    nib = (val >> (4 * n)) & 15
                    f = nib.astype(jnp.float32) * (1.0 / 15.0)
                    plsc.store_scatter(out_v, [out_base + (8 * w + n)], f)
            return carry2

        lax.fori_loop(0, CHUNK // 16, group_body, 0)
        pltpu.sync_copy(out_v, out_hbm.at[pl.ds(base * N_ZONAS, CHUNK * N_ZONAS)])
        return carry

    lax.fori_loop(0, CHUNKS, chunk_body, 0)


@jax.jit
def kernel(token_ids, tabla_cuant):
    flat_ids = token_ids.reshape(-1)
    padded = jnp.pad(tabla_cuant, ((0, 0), (0, 6)))
    table_i32 = lax.bitcast_convert_type(
        padded.reshape(VOCAB, ROW_WORDS, 4), jnp.int32
    )
    mesh = plsc.VectorSubcoreMesh(core_axis_name="c", subcore_axis_name="s")
    out = pl.kernel(
        _sc_kernel,
        out_type=jax.ShapeDtypeStruct((TOKENS * N_ZONAS,), jnp.float32),
        mesh=mesh,
        scratch_types=[
            pltpu.VMEM((CHUNK,), jnp.int32),
            pltpu.VMEM((GATHER_SPLIT, 128), jnp.int32),
            pltpu.VMEM((CHUNK, ROW_WORDS), jnp.int32),
            pltpu.VMEM((CHUNK * N_ZONAS,), jnp.float32),
            pltpu.SemaphoreType.DMA,
        ],
        compiler_params=pltpu.CompilerParams(
            needs_layout_passes=False, use_tc_tiling_on_sc=False
        ),
    )(table_i32, flat_ids)
    return out.reshape(B, L, N_ZONAS)
